# baseline (speedup 1.0000x reference)
import functools

import jax
import jax.numpy as jnp
from jax import lax
from jax.experimental import pallas as pl
from jax.experimental.pallas import tpu as pltpu
from jax.experimental.pallas import tpu_sc as plsc

_SH = 9
_NC, _NS, _L = 2, 16, 16
_NW = _NC * _NS
_IDXCHUNK = 128


def _build(B, V, C):
    n_chunks = B // (_NW * C)
    mesh = plsc.VectorSubcoreMesh(core_axis_name="c", subcore_axis_name="s")

    @functools.partial(
        pl.kernel,
        out_type=(jax.ShapeDtypeStruct((B,), jnp.float32),
                  jax.ShapeDtypeStruct((B, 3), jnp.float32)),
        mesh=mesh,
        compiler_params=pltpu.CompilerParams(needs_layout_passes=False,
                                             use_tc_tiling_on_sc=False),
        scratch_types=[
            pltpu.VMEM((C, 3), jnp.float32),
            pltpu.VMEM((C, 3), jnp.float32),
            pltpu.VMEM((C,), jnp.int32),
            pltpu.VMEM((27, C), jnp.float32),
            pltpu.VMEM((C,), jnp.float32),
            pltpu.VMEM((C,), jnp.float32),
            pltpu.VMEM((C, 3), jnp.float32),
            pltpu.SemaphoreType.DMA,
        ],
    )
    def k(pos_hbm, dir_hbm, den_hbm, *rest):
        csl_hbm = rest[:3]
        (dens_out, col_out,
         pos_v, dir_v, idx_v, coef_v, deng_v, den_v, col_v, sem) = rest[3:]
        wid = lax.axis_index("s") * _NC + lax.axis_index("c")
        per_w = B // _NW
        iota = lax.iota(jnp.int32, _L)
        cols = [jnp.full((_L,), c, jnp.int32) for c in range(3)]

        def do_chunk(t, carry):
            base = wid * per_w + t * C
            cp = pltpu.async_copy(pos_hbm.at[pl.ds(base, C)], pos_v, sem)
            cd = pltpu.async_copy(dir_hbm.at[pl.ds(base, C)], dir_v, sem)
            cp.wait()
            cd.wait()

            def compute_idx(i, carry2):
                s = iota + i * _L
                px = plsc.load_gather(pos_v, [s, cols[0]])
                py = plsc.load_gather(pos_v, [s, cols[1]])
                pz = plsc.load_gather(pos_v, [s, cols[2]])
                gx = jnp.clip(jnp.clip((px + 1.0) * 0.5, 0.0, 1.0) * 128.0,
                              0.0, 127.0)
                gy = jnp.clip(jnp.clip((py + 1.0) * 0.5, 0.0, 1.0) * 128.0,
                              0.0, 127.0)
                gz = jnp.clip(jnp.clip((pz + 1.0) * 0.5, 0.0, 1.0) * 128.0,
                              0.0, 127.0)
                fidx = gx * 16384.0 + gy * 128.0 + gz
                idx_v[pl.ds(i * _L, _L)] = fidx.astype(jnp.int32)
                return carry2

            lax.fori_loop(0, C // _L, compute_idx, 0)

            for j in range(C // _IDXCHUNK):
                sl = pl.ds(j * _IDXCHUNK, _IDXCHUNK)
                idx_sl = idx_v.at[sl]
                descs = [pltpu.async_copy(
                    den_hbm.at[idx_sl], deng_v.at[sl], sem)]
                for kc in range(27):
                    descs.append(pltpu.async_copy(
                        csl_hbm[kc % 3].at[kc // 3].at[idx_sl],
                        coef_v.at[kc, sl], sem))
                for dsc in descs:
                    dsc.wait()

            def compute_out(i, carry2):
                s = iota + i * _L
                sl = pl.ds(i * _L, _L)
                dx = plsc.load_gather(dir_v, [s, cols[0]])
                dy = plsc.load_gather(dir_v, [s, cols[1]])
                dz = plsc.load_gather(dir_v, [s, cols[2]])
                basis = [
                    jnp.full((_L,), 0.28209479177387814, jnp.float32),
                    0.4886025119029199 * dy,
                    0.4886025119029199 * dz,
                    0.4886025119029199 * dx,
                    1.0925484305920792 * (dx * dy),
                    1.0925484305920792 * (dy * dz),
                    0.31539156525252005 * (3.0 * (dz * dz) - 1.0),
                    1.0925484305920792 * (dx * dz),
                    0.5462742152960396 * (dx * dx - dy * dy),
                ]
                for c in range(3):
                    acc = basis[0] * coef_v[c, sl]
                    for kk in range(1, _SH):
                        acc = acc + basis[kk] * coef_v[kk * 3 + c, sl]
                    col = 1.0 / (1.0 + jnp.exp(-acc))
                    plsc.store_scatter(col_v, [s, cols[c]], col)
                den = deng_v[sl]
                den_v[sl] = jnp.maximum(den, 0.0)
                return carry2

            lax.fori_loop(0, C // _L, compute_out, 0)

            co1 = pltpu.async_copy(den_v, dens_out.at[pl.ds(base, C)], sem)
            co2 = pltpu.async_copy(col_v, col_out.at[pl.ds(base, C)], sem)
            co1.wait()
            co2.wait()
            return carry

        lax.fori_loop(0, n_chunks, do_chunk, 0)

    return k


def kernel(positions, directions, density_grid, sh_grid):
    B = positions.shape[0]
    V = sh_grid.shape[0]
    t = jnp.transpose(sh_grid, (1, 2, 0))
    slabs = [t[:, c, :] for c in range(3)]
    return _build(B, V, 1024)(positions, directions, density_grid, *slabs)


# speedup vs baseline: 4.3897x; 4.3897x over previous
import functools

import jax
import jax.numpy as jnp
from jax import lax
from jax.experimental import pallas as pl
from jax.experimental.pallas import tpu as pltpu
from jax.experimental.pallas import tpu_sc as plsc

_SH = 9
_NC, _NS, _L = 2, 16, 16
_NW = _NC * _NS
_IDXCHUNK = 128


def _build(B, V, C):
    n_chunks = B // (_NW * C)
    mesh = plsc.VectorSubcoreMesh(core_axis_name="c", subcore_axis_name="s")

    @functools.partial(
        pl.kernel,
        out_type=(jax.ShapeDtypeStruct((B,), jnp.float32),
                  jax.ShapeDtypeStruct((B, 3), jnp.float32)),
        mesh=mesh,
        compiler_params=pltpu.CompilerParams(needs_layout_passes=False,
                                             use_tc_tiling_on_sc=False),
        scratch_types=[
            pltpu.VMEM((C, 3), jnp.float32),
            pltpu.VMEM((C, 3), jnp.float32),
            pltpu.VMEM((C,), jnp.int32),
            pltpu.VMEM((27, C), jnp.float32),
            pltpu.VMEM((C,), jnp.float32),
            pltpu.VMEM((C,), jnp.float32),
            pltpu.VMEM((C, 3), jnp.float32),
            pltpu.SemaphoreType.DMA,
            pltpu.SemaphoreType.DMA,
        ],
    )
    def k(pos_hbm, dir_hbm, den_hbm, *rest):
        plane_hbm = rest[:27]
        (dens_out, col_out,
         pos_v, dir_v, idx_v, coef_v, deng_v, den_v, col_v,
         sem, sem2) = rest[27:]
        wid = lax.axis_index("s") * _NC + lax.axis_index("c")
        per_w = B // _NW
        iota = lax.iota(jnp.int32, _L)
        cols = [jnp.full((_L,), c, jnp.int32) for c in range(3)]

        def do_chunk(t, carry):
            base = wid * per_w + t * C
            cp = pltpu.async_copy(pos_hbm.at[pl.ds(base, C)], pos_v, sem)
            cd = pltpu.async_copy(dir_hbm.at[pl.ds(base, C)], dir_v, sem2)
            cp.wait()

            def compute_idx(i, carry2):
                s = iota + i * _L
                px = plsc.load_gather(pos_v, [s, cols[0]])
                py = plsc.load_gather(pos_v, [s, cols[1]])
                pz = plsc.load_gather(pos_v, [s, cols[2]])
                gx = jnp.clip(jnp.clip((px + 1.0) * 0.5, 0.0, 1.0) * 128.0,
                              0.0, 127.0)
                gy = jnp.clip(jnp.clip((py + 1.0) * 0.5, 0.0, 1.0) * 128.0,
                              0.0, 127.0)
                gz = jnp.clip(jnp.clip((pz + 1.0) * 0.5, 0.0, 1.0) * 128.0,
                              0.0, 127.0)
                fidx = gx * 16384.0 + gy * 128.0 + gz
                idx_v[pl.ds(i * _L, _L)] = fidx.astype(jnp.int32)
                return carry2

            lax.fori_loop(0, C // _L, compute_idx, 0)

            def fire(j):
                sl = pl.ds(j * _IDXCHUNK, _IDXCHUNK)
                idx_sl = idx_v.at[sl]
                descs = [pltpu.async_copy(
                    den_hbm.at[idx_sl], deng_v.at[sl], sem)]
                for kc in range(27):
                    descs.append(pltpu.async_copy(
                        plane_hbm[kc].at[idx_sl], coef_v.at[kc, sl], sem))
                return descs

            descs = fire(0)
            for j in range(1, C // _IDXCHUNK):
                nxt = fire(j)
                for dsc in descs:
                    dsc.wait()
                descs = nxt
            for dsc in descs:
                dsc.wait()
            cd.wait()

            def compute_out(i, carry2):
                s = iota + i * _L
                sl = pl.ds(i * _L, _L)
                dx = plsc.load_gather(dir_v, [s, cols[0]])
                dy = plsc.load_gather(dir_v, [s, cols[1]])
                dz = plsc.load_gather(dir_v, [s, cols[2]])
                basis = [
                    jnp.full((_L,), 0.28209479177387814, jnp.float32),
                    0.4886025119029199 * dy,
                    0.4886025119029199 * dz,
                    0.4886025119029199 * dx,
                    1.0925484305920792 * (dx * dy),
                    1.0925484305920792 * (dy * dz),
                    0.31539156525252005 * (3.0 * (dz * dz) - 1.0),
                    1.0925484305920792 * (dx * dz),
                    0.5462742152960396 * (dx * dx - dy * dy),
                ]
                for c in range(3):
                    acc = basis[0] * coef_v[c, sl]
                    for kk in range(1, _SH):
                        acc = acc + basis[kk] * coef_v[kk * 3 + c, sl]
                    col = 1.0 / (1.0 + jnp.exp(-acc))
                    plsc.store_scatter(col_v, [s, cols[c]], col)
                den = deng_v[sl]
                den_v[sl] = jnp.maximum(den, 0.0)
                return carry2

            lax.fori_loop(0, C // _L, compute_out, 0)

            co1 = pltpu.async_copy(den_v, dens_out.at[pl.ds(base, C)], sem)
            co2 = pltpu.async_copy(col_v, col_out.at[pl.ds(base, C)], sem)
            co1.wait()
            co2.wait()
            return carry

        lax.fori_loop(0, n_chunks, do_chunk, 0)

    return k


def kernel(positions, directions, density_grid, sh_grid):
    B = positions.shape[0]
    V = sh_grid.shape[0]
    planes = [sh_grid[:, kk, c] for kk in range(9) for c in range(3)]
    return _build(B, V, 1024)(positions, directions, density_grid, *planes)


# speedup vs baseline: 6.6579x; 1.5167x over previous
import functools

import jax
import jax.numpy as jnp
from jax import lax
from jax.experimental import pallas as pl
from jax.experimental.pallas import tpu as pltpu
from jax.experimental.pallas import tpu_sc as plsc

_SH = 9
_NC, _NS, _L = 2, 16, 16
_NW = _NC * _NS
_IDXCHUNK = 128


def _build(B, V, C):
    n_chunks = B // (_NW * C)
    mesh = plsc.VectorSubcoreMesh(core_axis_name="c", subcore_axis_name="s")

    @functools.partial(
        pl.kernel,
        out_type=(jax.ShapeDtypeStruct((B,), jnp.float32),
                  jax.ShapeDtypeStruct((B, 3), jnp.float32)),
        mesh=mesh,
        compiler_params=pltpu.CompilerParams(needs_layout_passes=False,
                                             use_tc_tiling_on_sc=False),
        scratch_types=[
            pltpu.VMEM((3, C), jnp.float32),
            pltpu.VMEM((3, C), jnp.float32),
            pltpu.VMEM((C,), jnp.int32),
            pltpu.VMEM((27, C), jnp.float32),
            pltpu.VMEM((C,), jnp.float32),
            pltpu.VMEM((C,), jnp.float32),
            pltpu.VMEM((C, 3), jnp.float32),
            pltpu.SemaphoreType.DMA,
            pltpu.SemaphoreType.DMA,
        ],
    )
    def k(px_hbm, py_hbm, pz_hbm, dx_hbm, dy_hbm, dz_hbm, den_hbm, *rest):
        plane_hbm = rest[:27]
        (dens_out, col_out,
         pos_v, dir_v, idx_v, coef_v, deng_v, den_v, col_v,
         sem, sem2) = rest[27:]
        wid = lax.axis_index("s") * _NC + lax.axis_index("c")
        per_w = B // _NW
        iota = lax.iota(jnp.int32, _L)
        cols = [jnp.full((_L,), c, jnp.int32) for c in range(3)]

        def do_chunk(t, carry):
            base = wid * per_w + t * C
            sl_in = pl.ds(base, C)
            cps = [pltpu.async_copy(px_hbm.at[sl_in], pos_v.at[0], sem),
                   pltpu.async_copy(py_hbm.at[sl_in], pos_v.at[1], sem),
                   pltpu.async_copy(pz_hbm.at[sl_in], pos_v.at[2], sem)]
            cds = [pltpu.async_copy(dx_hbm.at[sl_in], dir_v.at[0], sem2),
                   pltpu.async_copy(dy_hbm.at[sl_in], dir_v.at[1], sem2),
                   pltpu.async_copy(dz_hbm.at[sl_in], dir_v.at[2], sem2)]
            for cp in cps:
                cp.wait()

            def compute_idx(i, carry2):
                sl2 = pl.ds(i * _L, _L)
                px = pos_v[0, sl2]
                py = pos_v[1, sl2]
                pz = pos_v[2, sl2]
                gx = jnp.clip(jnp.clip((px + 1.0) * 0.5, 0.0, 1.0) * 128.0,
                              0.0, 127.0)
                gy = jnp.clip(jnp.clip((py + 1.0) * 0.5, 0.0, 1.0) * 128.0,
                              0.0, 127.0)
                gz = jnp.clip(jnp.clip((pz + 1.0) * 0.5, 0.0, 1.0) * 128.0,
                              0.0, 127.0)
                fidx = gx * 16384.0 + gy * 128.0 + gz
                idx_v[sl2] = fidx.astype(jnp.int32)
                return carry2

            lax.fori_loop(0, C // _L, compute_idx, 0)

            def fire(j):
                sl = pl.ds(j * _IDXCHUNK, _IDXCHUNK)
                idx_sl = idx_v.at[sl]
                descs = [pltpu.async_copy(
                    den_hbm.at[idx_sl], deng_v.at[sl], sem)]
                for kc in range(27):
                    descs.append(pltpu.async_copy(
                        plane_hbm[kc].at[idx_sl], coef_v.at[kc, sl], sem))
                return descs

            descs = fire(0)
            for j in range(1, C // _IDXCHUNK):
                nxt = fire(j)
                for dsc in descs:
                    dsc.wait()
                descs = nxt
            for dsc in descs:
                dsc.wait()
            for cd in cds:
                cd.wait()

            def compute_out(i, carry2):
                s = iota + i * _L
                sl = pl.ds(i * _L, _L)
                dx = dir_v[0, sl]
                dy = dir_v[1, sl]
                dz = dir_v[2, sl]
                basis = [
                    jnp.full((_L,), 0.28209479177387814, jnp.float32),
                    0.4886025119029199 * dy,
                    0.4886025119029199 * dz,
                    0.4886025119029199 * dx,
                    1.0925484305920792 * (dx * dy),
                    1.0925484305920792 * (dy * dz),
                    0.31539156525252005 * (3.0 * (dz * dz) - 1.0),
                    1.0925484305920792 * (dx * dz),
                    0.5462742152960396 * (dx * dx - dy * dy),
                ]
                for c in range(3):
                    acc = basis[0] * coef_v[c, sl]
                    for kk in range(1, _SH):
                        acc = acc + basis[kk] * coef_v[kk * 3 + c, sl]
                    col = 1.0 / (1.0 + jnp.exp(-acc))
                    plsc.store_scatter(col_v, [s, cols[c]], col)
                den = deng_v[sl]
                den_v[sl] = jnp.maximum(den, 0.0)
                return carry2

            lax.fori_loop(0, C // _L, compute_out, 0)

            co1 = pltpu.async_copy(den_v, dens_out.at[pl.ds(base, C)], sem)
            co2 = pltpu.async_copy(col_v, col_out.at[pl.ds(base, C)], sem)
            co1.wait()
            co2.wait()
            return carry

        lax.fori_loop(0, n_chunks, do_chunk, 0)

    return k


def kernel(positions, directions, density_grid, sh_grid):
    B = positions.shape[0]
    V = sh_grid.shape[0]
    planes = [sh_grid[:, kk, c] for kk in range(9) for c in range(3)]
    return _build(B, V, 1024)(
        positions[:, 0], positions[:, 1], positions[:, 2],
        directions[:, 0], directions[:, 1], directions[:, 2],
        density_grid, *planes)


# speedup vs baseline: 8.3807x; 1.2588x over previous
import functools

import jax
import jax.numpy as jnp
from jax import lax
from jax.experimental import pallas as pl
from jax.experimental.pallas import tpu as pltpu
from jax.experimental.pallas import tpu_sc as plsc

_SH = 9
_NC, _NS, _L = 2, 16, 16
_NW = _NC * _NS
_IDXCHUNK = 128


def _build(B, V, C):
    n_chunks = B // (_NW * C)
    mesh = plsc.VectorSubcoreMesh(core_axis_name="c", subcore_axis_name="s")

    @functools.partial(
        pl.kernel,
        out_type=(jax.ShapeDtypeStruct((B,), jnp.float32),
                  jax.ShapeDtypeStruct((3, B), jnp.float32)),
        mesh=mesh,
        compiler_params=pltpu.CompilerParams(needs_layout_passes=False,
                                             use_tc_tiling_on_sc=False),
        scratch_types=[
            pltpu.VMEM((3, C), jnp.float32),
            pltpu.VMEM((3, C), jnp.float32),
            pltpu.VMEM((C,), jnp.int32),
            pltpu.VMEM((27, C), jnp.float32),
            pltpu.VMEM((C,), jnp.float32),
            pltpu.VMEM((C,), jnp.float32),
            pltpu.VMEM((3, C), jnp.float32),
            pltpu.SemaphoreType.DMA,
            pltpu.SemaphoreType.DMA,
        ],
    )
    def k(px_hbm, py_hbm, pz_hbm, dx_hbm, dy_hbm, dz_hbm, den_hbm, *rest):
        plane_hbm = rest[:27]
        (dens_out, col_out,
         pos_v, dir_v, idx_v, coef_v, deng_v, den_v, col_v,
         sem, sem2) = rest[27:]
        wid = lax.axis_index("s") * _NC + lax.axis_index("c")
        per_w = B // _NW

        def do_chunk(t, carry):
            base = wid * per_w + t * C
            sl_in = pl.ds(base, C)
            cps = [pltpu.async_copy(px_hbm.at[sl_in], pos_v.at[0], sem),
                   pltpu.async_copy(py_hbm.at[sl_in], pos_v.at[1], sem),
                   pltpu.async_copy(pz_hbm.at[sl_in], pos_v.at[2], sem)]
            cds = [pltpu.async_copy(dx_hbm.at[sl_in], dir_v.at[0], sem2),
                   pltpu.async_copy(dy_hbm.at[sl_in], dir_v.at[1], sem2),
                   pltpu.async_copy(dz_hbm.at[sl_in], dir_v.at[2], sem2)]
            for cp in cps:
                cp.wait()

            def compute_idx(i, carry2):
                sl2 = pl.ds(i * _L, _L)
                px = pos_v[0, sl2]
                py = pos_v[1, sl2]
                pz = pos_v[2, sl2]
                gx = jnp.clip(jnp.clip((px + 1.0) * 0.5, 0.0, 1.0) * 128.0,
                              0.0, 127.0)
                gy = jnp.clip(jnp.clip((py + 1.0) * 0.5, 0.0, 1.0) * 128.0,
                              0.0, 127.0)
                gz = jnp.clip(jnp.clip((pz + 1.0) * 0.5, 0.0, 1.0) * 128.0,
                              0.0, 127.0)
                fidx = gx * 16384.0 + gy * 128.0 + gz
                idx_v[sl2] = fidx.astype(jnp.int32)
                return carry2

            lax.fori_loop(0, C // _L, compute_idx, 0)

            def fire(j):
                sl = pl.ds(j * _IDXCHUNK, _IDXCHUNK)
                idx_sl = idx_v.at[sl]
                descs = [pltpu.async_copy(
                    den_hbm.at[idx_sl], deng_v.at[sl], sem)]
                for kc in range(27):
                    descs.append(pltpu.async_copy(
                        plane_hbm[kc].at[idx_sl], coef_v.at[kc, sl], sem))
                return descs

            descs = fire(0)
            for j in range(1, C // _IDXCHUNK):
                nxt = fire(j)
                for dsc in descs:
                    dsc.wait()
                descs = nxt
            for dsc in descs:
                dsc.wait()
            for cd in cds:
                cd.wait()

            def compute_out(i, carry2):
                sl = pl.ds(i * _L, _L)
                dx = dir_v[0, sl]
                dy = dir_v[1, sl]
                dz = dir_v[2, sl]
                basis = [
                    jnp.full((_L,), 0.28209479177387814, jnp.float32),
                    0.4886025119029199 * dy,
                    0.4886025119029199 * dz,
                    0.4886025119029199 * dx,
                    1.0925484305920792 * (dx * dy),
                    1.0925484305920792 * (dy * dz),
                    0.31539156525252005 * (3.0 * (dz * dz) - 1.0),
                    1.0925484305920792 * (dx * dz),
                    0.5462742152960396 * (dx * dx - dy * dy),
                ]
                for c in range(3):
                    acc = basis[0] * coef_v[c, sl]
                    for kk in range(1, _SH):
                        acc = acc + basis[kk] * coef_v[kk * 3 + c, sl]
                    col = 1.0 / (1.0 + jnp.exp(-acc))
                    col_v[c, sl] = col
                den = deng_v[sl]
                den_v[sl] = jnp.maximum(den, 0.0)
                return carry2

            lax.fori_loop(0, C // _L, compute_out, 0)

            outs = [pltpu.async_copy(den_v, dens_out.at[sl_in], sem),
                    pltpu.async_copy(col_v.at[0], col_out.at[0].at[sl_in], sem),
                    pltpu.async_copy(col_v.at[1], col_out.at[1].at[sl_in], sem),
                    pltpu.async_copy(col_v.at[2], col_out.at[2].at[sl_in], sem)]
            for co in outs:
                co.wait()
            return carry

        lax.fori_loop(0, n_chunks, do_chunk, 0)

    return k


def kernel(positions, directions, density_grid, sh_grid):
    B = positions.shape[0]
    V = sh_grid.shape[0]
    planes = [sh_grid[:, kk, c] for kk in range(9) for c in range(3)]
    dens, col_t = _build(B, V, 1024)(
        positions[:, 0], positions[:, 1], positions[:, 2],
        directions[:, 0], directions[:, 1], directions[:, 2],
        density_grid, *planes)
    return dens, jnp.transpose(col_t)
